# baseline (speedup 1.0000x reference)
import functools

import jax
import jax.numpy as jnp
from jax import lax
from jax.experimental import pallas as pl
from jax.experimental.pallas import tpu as pltpu
from jax.experimental.pallas import tpu_sc as plsc


def _sc_partials(dst2d, src2d, feat_table):
    NCH, CH = dst2d.shape
    N, D = feat_table.shape
    NC, NS = 2, 16
    PC = NCH // NC
    TPC = PC // NS
    NA = N + 8
    RB = 80
    NRC = N // RB
    TRC = -(-NRC // NS)
    DT = N // 10

    mesh = plsc.VectorSubcoreMesh(core_axis_name="c", subcore_axis_name="s")

    @functools.partial(
        pl.kernel,
        out_type=(
            jax.ShapeDtypeStruct((NC, N, D), jnp.float32),
            jax.ShapeDtypeStruct((NC * N,), jnp.float32),
        ),
        mesh=mesh,
        scratch_types=(
            pltpu.VMEM((TPC // 2, CH), jnp.int32),
            pltpu.VMEM((TPC // 2, CH), jnp.int32),
            pltpu.VMEM((2, CH, D), jnp.float32),
            pltpu.VMEM((CH,), jnp.float32),
            pltpu.VMEM((1024,), jnp.float32),
            pltpu.VMEM_SHARED((NA, D), jnp.float32),
            pltpu.VMEM_SHARED((NA,), jnp.float32),
            pltpu.SemaphoreType.DMA,
            pltpu.SemaphoreType.DMA,
            pltpu.SemaphoreType.DMA,
        ),
    )
    def sc_kernel(dst_ref, src_ref, feat_ref, part_ref, degp_ref,
                  srcb, dstb, rows2, onesv, zv, agg_sh, deg_sh,
                  sem_g, sem_s, sem_d):
        c = lax.axis_index("c")
        s = lax.axis_index("s")

        zero16 = jnp.zeros((16,), jnp.float32)
        one16 = jnp.ones((16,), jnp.float32)
        for j in range(CH // 16):
            onesv[pl.ds(j * 16, 16)] = one16
        for j in range(1024 // 16):
            zv[pl.ds(j * 16, 16)] = zero16

        def zrow(i, carry):
            for j in range(D // 16):
                rows2[0, i, pl.ds(j * 16, 16)] = zero16
            return carry
        lax.fori_loop(0, CH, zrow, 0)

        def zchunk(t, carry):
            idx = s + NS * t

            @pl.when(idx < NRC)
            def _():
                pltpu.sync_copy(rows2.at[0, pl.ds(0, RB)],
                                agg_sh.at[pl.ds(idx * RB, RB)])
            return carry
        lax.fori_loop(0, TRC, zchunk, 0)

        @pl.when(s < 10)
        def _():
            pltpu.sync_copy(zv.at[pl.ds(0, DT)], deg_sh.at[pl.ds(s * DT, DT)])

        plsc.subcore_barrier()

        glo = c * PC + s * TPC
        TPH = TPC // 2

        def start_gather(t, b):
            pltpu.async_copy(feat_ref.at[srcb.at[t]], rows2.at[b], sem_g)

        def wait_gather(t, b):
            pltpu.make_async_copy(feat_ref.at[srcb.at[t]], rows2.at[b],
                                  sem_g).wait()

        def start_scatter(t, b):
            pltpu.async_copy(rows2.at[b], agg_sh.at[dstb.at[t]], sem_s,
                             add=True)
            pltpu.async_copy(onesv, deg_sh.at[dstb.at[t]], sem_d, add=True)

        def wait_scatter(t, b):
            pltpu.make_async_copy(rows2.at[b], agg_sh.at[dstb.at[t]],
                                  sem_s).wait()
            pltpu.make_async_copy(onesv, deg_sh.at[dstb.at[t]], sem_d).wait()

        for ph in range(2):
            pltpu.sync_copy(src_ref.at[pl.ds(glo + ph * TPH, TPH)], srcb)
            pltpu.sync_copy(dst_ref.at[pl.ds(glo + ph * TPH, TPH)], dstb)
            start_gather(0, 0)

            def chunk_body(t, carry):
                b = lax.rem(t, 2)
                wait_gather(t, b)

                @pl.when(t >= 1)
                def _():
                    wait_scatter(t - 1, 1 - b)

                @pl.when(t + 1 < TPH)
                def _():
                    start_gather(t + 1, 1 - b)

                start_scatter(t, b)
                return carry
            lax.fori_loop(0, TPH, chunk_body, 0)
            wait_scatter(TPH - 1, (TPH - 1) % 2)

        plsc.subcore_barrier()

        def wchunk(t, carry):
            idx = s + NS * t

            @pl.when(idx < NRC)
            def _():
                r0 = idx * RB
                pltpu.sync_copy(agg_sh.at[pl.ds(r0, RB)],
                                rows2.at[0, pl.ds(0, RB)])
                pltpu.sync_copy(rows2.at[0, pl.ds(0, RB)],
                                part_ref.at[c, pl.ds(r0, RB)])
            return carry
        lax.fori_loop(0, TRC, wchunk, 0)

        @pl.when(s < 10)
        def _():
            pltpu.sync_copy(deg_sh.at[pl.ds(s * DT, DT)], zv.at[pl.ds(0, DT)])
            pltpu.sync_copy(zv.at[pl.ds(0, DT)],
                            degp_ref.at[pl.ds(c * N + s * DT, DT)])

    return sc_kernel(dst2d, src2d, feat_table)


def _combine(part, degp, feat_table):
    N, D = feat_table.shape
    R = 1000

    def body(p_ref, d_ref, f_ref, o_ref):
        num = p_ref[0] + p_ref[1] + f_ref[...]
        deg = d_ref[0] + d_ref[1] + 1.0
        o_ref[...] = num / deg

    return pl.pallas_call(
        body,
        grid=(N // R,),
        in_specs=[
            pl.BlockSpec((2, R, D), lambda i: (0, i, 0)),
            pl.BlockSpec((2, R, 1), lambda i: (0, i, 0)),
            pl.BlockSpec((R, D), lambda i: (i, 0)),
        ],
        out_specs=pl.BlockSpec((R, D), lambda i: (i, 0)),
        out_shape=jax.ShapeDtypeStruct((N, D), jnp.float32),
    )(part, degp.reshape(2, N, 1), feat_table)


def kernel(nodes, edge_index, feat_table):
    CH = 128
    E = edge_index.shape[1]
    N = feat_table.shape[0]
    NCH = -(-E // (CH * 32 * 8)) * 32 * 8
    EP = NCH * CH - E
    dst2d = jnp.concatenate(
        [edge_index[0], jnp.full((EP,), N, jnp.int32)]).reshape(NCH, CH)
    src2d = jnp.concatenate(
        [edge_index[1], jnp.zeros((EP,), jnp.int32)]).reshape(NCH, CH)
    part, degp = _sc_partials(dst2d, src2d, feat_table)
    return _combine(part, degp, feat_table)
